# baseline (speedup 1.0000x reference)
import functools

import jax
import jax.numpy as jnp
from jax import lax
from jax.experimental import pallas as pl
from jax.experimental.pallas import tpu as pltpu
from jax.experimental.pallas import tpu_sc as plsc

S = 2048
D_MODEL = 768
D_FF = 3072
E = 8
TOPK = 2
TM = 256
MPAD = S * TOPK + E * TM
NT = MPAD // TM
NF = 2
FFB = D_FF // NF

NC = 2
NS = 16
NW = NC * NS
LANES = 16

GROWS = MPAD // NW
CROWS = S // NW


def _wid():
    return lax.axis_index("s") * NC + lax.axis_index("c")


def _sc_dispatch_body(src_hbm, x_hbm, xs_hbm, idx_v, row_v, sem):
    base = _wid() * GROWS

    def chunk(c, carry):
        b = base + c * LANES
        pltpu.sync_copy(src_hbm.at[pl.ds(b, LANES)], idx_v)
        pltpu.async_copy(x_hbm.at[idx_v], row_v, sem).wait()
        pltpu.sync_copy(row_v, xs_hbm.at[pl.ds(b, LANES)])
        return carry

    lax.fori_loop(0, GROWS // LANES, chunk, 0)


@functools.cache
def _sc_dispatch_kernel():
    return pl.kernel(
        _sc_dispatch_body,
        out_type=jax.ShapeDtypeStruct((MPAD, D_MODEL), jnp.float32),
        mesh=plsc.VectorSubcoreMesh(core_axis_name="c", subcore_axis_name="s"),
        scratch_types=[
            pltpu.VMEM((LANES,), jnp.int32),
            pltpu.VMEM((LANES, D_MODEL), jnp.float32),
            pltpu.SemaphoreType.DMA,
        ],
    )


def _sc_combine_body(p0_hbm, p1_hbm, ys_hbm, y_hbm, i0_v, i1_v, a_v, b_v, sem):
    base = _wid() * CROWS

    def chunk(c, carry):
        b = base + c * LANES
        pltpu.sync_copy(p0_hbm.at[pl.ds(b, LANES)], i0_v)
        pltpu.sync_copy(p1_hbm.at[pl.ds(b, LANES)], i1_v)
        pltpu.async_copy(ys_hbm.at[i0_v], a_v, sem).wait()
        pltpu.async_copy(ys_hbm.at[i1_v], b_v, sem).wait()

        def row(r, rc):
            for col in range(D_MODEL // LANES):
                sl = pl.ds(col * LANES, LANES)
                a_v[r, sl] = a_v[r, sl] + b_v[r, sl]
            return rc

        lax.fori_loop(0, LANES, row, 0)
        pltpu.sync_copy(a_v, y_hbm.at[pl.ds(b, LANES)])
        return carry

    lax.fori_loop(0, CROWS // LANES, chunk, 0)


@functools.cache
def _sc_combine_kernel():
    return pl.kernel(
        _sc_combine_body,
        out_type=jax.ShapeDtypeStruct((S, D_MODEL), jnp.float32),
        mesh=plsc.VectorSubcoreMesh(core_axis_name="c", subcore_axis_name="s"),
        scratch_types=[
            pltpu.VMEM((LANES,), jnp.int32),
            pltpu.VMEM((LANES,), jnp.int32),
            pltpu.VMEM((LANES, D_MODEL), jnp.float32),
            pltpu.VMEM((LANES, D_MODEL), jnp.float32),
            pltpu.SemaphoreType.DMA,
        ],
    )


def _ffn_body(g_ref, u_ref, xs_ref, w1_ref, b1_ref, w2_ref, b2_ref, ws_ref,
              ys_ref):
    i = pl.program_id(0)
    f = pl.program_id(1)

    @pl.when(i < u_ref[0])
    def _compute():
        xb = xs_ref[...].astype(jnp.bfloat16)
        w1 = w1_ref[0].astype(jnp.bfloat16)
        h = lax.dot_general(xb, w1, (((1,), (1,)), ((), ())),
                            preferred_element_type=jnp.float32)
        h = jnp.maximum(h + b1_ref[0, 0][None, :], 0.0).astype(jnp.bfloat16)
        w2 = w2_ref[0].astype(jnp.bfloat16)
        o = lax.dot_general(h, w2, (((1,), (1,)), ((), ())),
                            preferred_element_type=jnp.float32)
        b2 = jnp.where(f == 0, b2_ref[0, 0], 0.0)
        o = (o + b2[None, :]) * ws_ref[0, 0][:, None]

        @pl.when(f == 0)
        def _set():
            ys_ref[...] = o

        @pl.when(f != 0)
        def _acc():
            ys_ref[...] += o


@jax.jit
def _grouped_ffn(g, u, xs, w1, b1, w2, b2, ws):
    grid_spec = pltpu.PrefetchScalarGridSpec(
        num_scalar_prefetch=2,
        grid=(NT, NF),
        in_specs=[
            pl.BlockSpec((TM, D_MODEL), lambda i, f, g, u: (i, 0)),
            pl.BlockSpec((1, FFB, D_MODEL), lambda i, f, g, u: (g[i], f, 0)),
            pl.BlockSpec((1, 1, FFB), lambda i, f, g, u: (g[i], 0, f)),
            pl.BlockSpec((1, D_MODEL, FFB), lambda i, f, g, u: (g[i], 0, f)),
            pl.BlockSpec((1, 1, D_MODEL), lambda i, f, g, u: (g[i], 0, 0)),
            pl.BlockSpec((1, 1, TM), lambda i, f, g, u: (i, 0, 0)),
        ],
        out_specs=pl.BlockSpec((TM, D_MODEL), lambda i, f, g, u: (i, 0)),
    )
    return pl.pallas_call(
        _ffn_body,
        grid_spec=grid_spec,
        out_shape=jax.ShapeDtypeStruct((MPAD, D_MODEL), jnp.float32),
        compiler_params=pltpu.CompilerParams(
            dimension_semantics=("arbitrary", "arbitrary"),
        ),
    )(g, u, xs, w1, b1, w2, b2, ws)


def _routing(topk_probs, topk_idx):
    memb = (jax.nn.one_hot(topk_idx[:, 0], E, dtype=jnp.int32)
            + jax.nn.one_hot(topk_idx[:, 1], E, dtype=jnp.int32))
    cum = jnp.cumsum(memb, axis=0)
    counts = cum[-1]
    excl = cum - memb
    cnt_pad = ((counts + TM - 1) // TM) * TM
    bound = jnp.cumsum(cnt_pad)
    astart = bound - cnt_pad
    pos = astart[topk_idx] + jnp.take_along_axis(excl, topk_idx, axis=1)

    mflat = pos.reshape(-1)
    tok = jnp.arange(S * TOPK, dtype=jnp.int32) // TOPK
    src = jnp.zeros((MPAD,), jnp.int32).at[mflat].set(tok)
    ws = jnp.zeros((MPAD,), jnp.float32).at[mflat].set(topk_probs.reshape(-1))

    nused = (bound[-1] // TM).astype(jnp.int32)
    tile_start = jnp.arange(NT, dtype=jnp.int32) * TM
    g = jnp.searchsorted(bound, tile_start, side='right').astype(jnp.int32)
    g = jnp.where(jnp.arange(NT) < nused, jnp.minimum(g, E - 1),
                  jnp.minimum(g[jnp.maximum(nused - 1, 0)], E - 1))
    return src, ws, pos, g, nused


def kernel(x, gate_w, w1, b1, w2, b2):
    s, b, h = x.shape
    x_flat = x.reshape(s * b, h)

    logits = x_flat @ gate_w.T
    probs = jax.nn.softmax(logits, axis=-1)
    topk_probs, topk_idx = jax.lax.top_k(probs, TOPK)

    src, ws, pos, g, nused = _routing(topk_probs, topk_idx)

    xs = _sc_dispatch_kernel()(src, x_flat)
    ys = _grouped_ffn(
        g, nused.reshape(1), xs, w1,
        b1.reshape(E, 1, D_FF), w2, b2.reshape(E, 1, D_MODEL),
        ws.reshape(NT, 1, TM),
    )
    p0 = pos[:, 0]
    p1 = pos[:, 1]
    y_flat = _sc_combine_kernel()(p0, p1, ys)
    return y_flat.reshape(s, b, h)
